# Initial kernel scaffold; baseline (speedup 1.0000x reference)
import jax
import jax.numpy as jnp
from jax.experimental import pallas as pl


def kernel(question_ids, responses, item_embed, W_item, W_resp_w, W_resp_b):
    raise NotImplementedError("write your pallas kernel here")


# speedup vs baseline: 11.4545x; 11.4545x over previous
import functools

import jax
import jax.numpy as jnp
from jax import lax
from jax.experimental import pallas as pl
from jax.experimental.pallas import tpu as pltpu
from jax.experimental.pallas import tpu_sc as plsc

_NC = 2
_NS = 16
_NW = _NC * _NS

_G = 128
_KG = 5


def _sc_gather(table, ids2d, n_rows, h):
    rows_per_w = n_rows // _NW
    rows_per_it = _KG * _G
    n_it = rows_per_w // rows_per_it
    assert rows_per_w % rows_per_it == 0
    id_rows_per_w = rows_per_w // _G

    mesh = plsc.VectorSubcoreMesh(
        core_axis_name="c", subcore_axis_name="s",
        num_cores=_NC, num_subcores=_NS)

    @functools.partial(
        pl.kernel,
        out_type=jax.ShapeDtypeStruct((n_rows, h), jnp.float32),
        mesh=mesh,
        scratch_types=[
            pltpu.VMEM((id_rows_per_w, _G), jnp.int32),
            pltpu.VMEM((rows_per_it, h), jnp.float32),
            pltpu.SemaphoreType.DMA,
        ],
    )
    def gather_kernel(table_hbm, ids_hbm, out_hbm, idx_v, rows_v, sem):
        wid = lax.axis_index("s") * _NC + lax.axis_index("c")
        row0 = wid * rows_per_w

        pltpu.sync_copy(ids_hbm.at[wid], idx_v)

        def body(g, carry):
            copies = [
                pltpu.async_copy(
                    table_hbm.at[idx_v.at[g * _KG + j]],
                    rows_v.at[pl.ds(j * _G, _G)],
                    sem)
                for j in range(_KG)
            ]
            for c in copies:
                c.wait()
            pltpu.sync_copy(
                rows_v, out_hbm.at[pl.ds(row0 + g * rows_per_it, rows_per_it)])
            return carry

        lax.fori_loop(0, n_it, body, 0)

    return gather_kernel(table, ids2d)


def _tc_body(resp_ref, e_ref, wt_ref, wrt_ref, b_ref, out_ref, *, k):
    e = e_ref[...]
    acc = jnp.dot(e, wt_ref[...], preferred_element_type=jnp.float32)
    rf = resp_ref[...].astype(jnp.float32)
    kk = lax.broadcasted_iota(jnp.int32, (e.shape[0], k), 1).astype(jnp.float32)
    w = jnp.maximum(1.0 - jnp.abs(kk - rf) * (1.0 / (k - 1)), 0.0)
    acc = acc + jnp.dot(w, wrt_ref[...], preferred_element_type=jnp.float32)
    out_ref[...] = acc + b_ref[...]


def _tc_project(e, resp2d, w_item_t, w_resp_t, bias2d, tb):
    n, h = e.shape
    v = w_item_t.shape[1]
    k = w_resp_t.shape[0]
    grid = (n // tb,)
    return pl.pallas_call(
        functools.partial(_tc_body, k=k),
        grid=grid,
        in_specs=[
            pl.BlockSpec((tb, 1), lambda i: (i, 0)),
            pl.BlockSpec((tb, h), lambda i: (i, 0)),
            pl.BlockSpec((h, v), lambda i: (0, 0)),
            pl.BlockSpec((k, v), lambda i: (0, 0)),
            pl.BlockSpec((1, v), lambda i: (0, 0)),
        ],
        out_specs=pl.BlockSpec((tb, v), lambda i: (i, 0)),
        out_shape=jax.ShapeDtypeStruct((n, v), jnp.float32),
    )(resp2d, e, w_item_t, w_resp_t, bias2d)


def kernel(question_ids, responses, item_embed, W_item, W_resp_w, W_resp_b):
    b, s = question_ids.shape
    n = b * s
    q1, h = item_embed.shape
    v = W_item.shape[0]
    k = W_resp_w.shape[1]

    ids3d = question_ids.reshape(_NW, n // (_NW * _G), _G).astype(jnp.int32)
    e = _sc_gather(item_embed, ids3d, n, h)

    resp2d = responses.reshape(n, 1).astype(jnp.int32)
    out = _tc_project(e, resp2d, W_item.T, W_resp_w.T,
                      W_resp_b.reshape(1, v), tb=2048)
    return out.reshape(b, s, v)


# speedup vs baseline: 11.8218x; 1.0321x over previous
import functools

import jax
import jax.numpy as jnp
from jax import lax
from jax.experimental import pallas as pl
from jax.experimental.pallas import tpu as pltpu
from jax.experimental.pallas import tpu_sc as plsc

_NC = 2
_NS = 16
_NW = _NC * _NS

_G = 128
_KG = 5


def _sc_gather(table, ids2d, n_rows, h):
    rows_per_w = n_rows // _NW
    rows_per_it = _KG * _G
    n_it = rows_per_w // rows_per_it
    assert rows_per_w % rows_per_it == 0
    id_rows_per_w = rows_per_w // _G

    mesh = plsc.VectorSubcoreMesh(
        core_axis_name="c", subcore_axis_name="s",
        num_cores=_NC, num_subcores=_NS)

    @functools.partial(
        pl.kernel,
        out_type=jax.ShapeDtypeStruct((n_rows, h), jnp.float32),
        mesh=mesh,
        scratch_types=[
            pltpu.VMEM((id_rows_per_w, _G), jnp.int32),
            pltpu.VMEM((rows_per_it, h), jnp.float32),
            pltpu.SemaphoreType.DMA,
        ],
    )
    def gather_kernel(table_hbm, ids_hbm, out_hbm, idx_v, rows_v, sem):
        wid = lax.axis_index("s") * _NC + lax.axis_index("c")
        row0 = wid * rows_per_w

        pltpu.sync_copy(ids_hbm.at[wid], idx_v)

        def body(g, carry):
            copies = [
                pltpu.async_copy(
                    table_hbm.at[idx_v.at[g * _KG + j]],
                    rows_v.at[pl.ds(j * _G, _G)],
                    sem)
                for j in range(_KG)
            ]
            for c in copies:
                c.wait()
            pltpu.sync_copy(
                rows_v, out_hbm.at[pl.ds(row0 + g * rows_per_it, rows_per_it)])
            return carry

        lax.fori_loop(0, n_it, body, 0)

    return gather_kernel(table, ids2d)


def _tc_body(resp_ref, e_ref, wt_ref, wrt_ref, b_ref, out_ref, *, k):
    br, s = resp_ref.shape
    e = e_ref[...]
    acc = jnp.dot(e, wt_ref[...], preferred_element_type=jnp.float32)
    acc = acc + b_ref[...]
    c = lax.broadcasted_iota(jnp.int32, (k, 1), 0).astype(jnp.float32)
    inv = 1.0 / (k - 1)
    for row in range(br):
        rf = resp_ref[row, :].astype(jnp.float32).reshape(1, s)
        w = jnp.maximum(1.0 - jnp.abs(c - rf) * inv, 0.0)
        m = lax.dot_general(w, wrt_ref[...], (((0,), (0,)), ((), ())),
                            preferred_element_type=jnp.float32)
        out_ref[row] = acc[row * s:(row + 1) * s, :] + m


def _tc_project(e, resp, w_item_t, w_resp_t, bias2d, br):
    b, s = resp.shape
    h = e.shape[1]
    v = w_item_t.shape[1]
    k = w_resp_t.shape[0]
    grid = (b // br,)
    return pl.pallas_call(
        functools.partial(_tc_body, k=k),
        grid=grid,
        in_specs=[
            pl.BlockSpec((br, s), lambda i: (i, 0)),
            pl.BlockSpec((br * s, h), lambda i: (i, 0)),
            pl.BlockSpec((h, v), lambda i: (0, 0)),
            pl.BlockSpec((k, v), lambda i: (0, 0)),
            pl.BlockSpec((1, v), lambda i: (0, 0)),
        ],
        out_specs=pl.BlockSpec((br, s, v), lambda i: (i, 0, 0)),
        out_shape=jax.ShapeDtypeStruct((b, s, v), jnp.float32),
    )(resp, e, w_item_t, w_resp_t, bias2d)


def kernel(question_ids, responses, item_embed, W_item, W_resp_w, W_resp_b):
    b, s = question_ids.shape
    n = b * s
    q1, h = item_embed.shape
    v = W_item.shape[0]
    k = W_resp_w.shape[1]

    ids3d = question_ids.reshape(_NW, n // (_NW * _G), _G).astype(jnp.int32)
    e = _sc_gather(item_embed, ids3d, n, h)

    return _tc_project(e, responses.astype(jnp.int32), W_item.T, W_resp_w.T,
                       W_resp_b.reshape(1, v), br=8)


# speedup vs baseline: 13.4205x; 1.1352x over previous
import functools

import jax
import jax.numpy as jnp
from jax import lax
from jax.experimental import pallas as pl
from jax.experimental.pallas import tpu as pltpu
from jax.experimental.pallas import tpu_sc as plsc

_NC = 2
_NS = 16
_NW = _NC * _NS

_G = 128
_KG = 5


def _sc_gather(table, ids2d, n_rows, h):
    rows_per_w = n_rows // _NW
    rows_per_it = _KG * _G
    n_it = rows_per_w // rows_per_it
    assert rows_per_w % rows_per_it == 0
    id_rows_per_w = rows_per_w // _G

    mesh = plsc.VectorSubcoreMesh(
        core_axis_name="c", subcore_axis_name="s",
        num_cores=_NC, num_subcores=_NS)

    @functools.partial(
        pl.kernel,
        out_type=jax.ShapeDtypeStruct((n_rows, h), jnp.float32),
        mesh=mesh,
        scratch_types=[
            pltpu.VMEM((id_rows_per_w, _G), jnp.int32),
            pltpu.VMEM((rows_per_it, h), jnp.float32),
            pltpu.SemaphoreType.DMA,
        ],
    )
    def gather_kernel(table_hbm, ids_hbm, out_hbm, idx_v, rows_v, sem):
        wid = lax.axis_index("s") * _NC + lax.axis_index("c")
        row0 = wid * rows_per_w

        pltpu.sync_copy(ids_hbm.at[wid], idx_v)

        def body(g, carry):
            copies = [
                pltpu.async_copy(
                    table_hbm.at[idx_v.at[g * _KG + j]],
                    rows_v.at[pl.ds(j * _G, _G)],
                    sem)
                for j in range(_KG)
            ]
            for c in copies:
                c.wait()
            pltpu.sync_copy(
                rows_v, out_hbm.at[pl.ds(row0 + g * rows_per_it, rows_per_it)])
            return carry

        lax.fori_loop(0, n_it, body, 0)

    return gather_kernel(table, ids2d)


def _tc_body(resp_ref, e_ref, wt_ref, wrt_ref, b_ref, out_ref, *, k):
    sb = out_ref.shape[0]
    c = lax.broadcasted_iota(jnp.int32, (k, 1), 0).astype(jnp.float32)
    inv = 1.0 / (k - 1)
    bias = b_ref[...]
    for s in range(sb):
        es = e_ref[:, s, :]
        acc = lax.dot_general(wt_ref[...], es, (((0,), (1,)), ((), ())),
                              preferred_element_type=jnp.float32)
        rf = resp_ref[s, :].astype(jnp.float32).reshape(1, -1)
        w = jnp.maximum(1.0 - jnp.abs(c - rf) * inv, 0.0)
        acc = acc + lax.dot_general(wrt_ref[...], w, (((0,), (0,)), ((), ())),
                                    preferred_element_type=jnp.float32)
        out_ref[s] = acc + bias


def _tc_project(e3d, resp_t, w_item_t, w_resp_t, bias2d, sb, bb):
    b, s, h = e3d.shape
    v = w_item_t.shape[1]
    k = w_resp_t.shape[0]
    grid = (b // bb, s // sb)
    out = pl.pallas_call(
        functools.partial(_tc_body, k=k),
        grid=grid,
        in_specs=[
            pl.BlockSpec((sb, bb), lambda j, i: (i, j)),
            pl.BlockSpec((bb, sb, h), lambda j, i: (j, i, 0)),
            pl.BlockSpec((h, v), lambda j, i: (0, 0)),
            pl.BlockSpec((k, v), lambda j, i: (0, 0)),
            pl.BlockSpec((v, 1), lambda j, i: (0, 0)),
        ],
        out_specs=pl.BlockSpec((sb, v, bb), lambda j, i: (i, 0, j)),
        out_shape=jax.ShapeDtypeStruct((s, v, b), jnp.float32),
    )(resp_t, e3d, w_item_t, w_resp_t, bias2d)
    return jnp.transpose(out, (2, 0, 1))


def kernel(question_ids, responses, item_embed, W_item, W_resp_w, W_resp_b):
    b, s = question_ids.shape
    n = b * s
    q1, h = item_embed.shape
    v = W_item.shape[0]
    k = W_resp_w.shape[1]

    ids3d = question_ids.reshape(_NW, n // (_NW * _G), _G).astype(jnp.int32)
    e = _sc_gather(item_embed, ids3d, n, h)

    return _tc_project(e.reshape(b, s, h), responses.T.astype(jnp.int32),
                       W_item.T, W_resp_w.T, W_resp_b.reshape(v, 1),
                       sb=8, bb=128)


# speedup vs baseline: 14.0203x; 1.0447x over previous
import functools

import jax
import jax.numpy as jnp
from jax import lax
from jax.experimental import pallas as pl
from jax.experimental.pallas import tpu as pltpu
from jax.experimental.pallas import tpu_sc as plsc

_NC = 2
_NS = 16
_NW = _NC * _NS

_G = 128
_KG = 5


def _sc_gather(table, ids2d, n_rows, h):
    rows_per_w = n_rows // _NW
    rows_per_it = _KG * _G
    n_it = rows_per_w // rows_per_it
    assert rows_per_w % rows_per_it == 0
    id_rows_per_w = rows_per_w // _G

    mesh = plsc.VectorSubcoreMesh(
        core_axis_name="c", subcore_axis_name="s",
        num_cores=_NC, num_subcores=_NS)

    @functools.partial(
        pl.kernel,
        out_type=jax.ShapeDtypeStruct((n_rows, h), jnp.float32),
        mesh=mesh,
        scratch_types=[
            pltpu.VMEM((id_rows_per_w, _G), jnp.int32),
            pltpu.VMEM((rows_per_it, h), jnp.float32),
            pltpu.SemaphoreType.DMA,
        ],
    )
    def gather_kernel(table_hbm, ids_hbm, out_hbm, idx_v, rows_v, sem):
        wid = lax.axis_index("s") * _NC + lax.axis_index("c")
        row0 = wid * rows_per_w

        pltpu.sync_copy(ids_hbm.at[wid], idx_v)

        def body(g, carry):
            copies = [
                pltpu.async_copy(
                    table_hbm.at[idx_v.at[g * _KG + j]],
                    rows_v.at[pl.ds(j * _G, _G)],
                    sem)
                for j in range(_KG)
            ]
            for c in copies:
                c.wait()
            pltpu.sync_copy(
                rows_v, out_hbm.at[pl.ds(row0 + g * rows_per_it, rows_per_it)])
            return carry

        lax.fori_loop(0, n_it, body, 0)

    return gather_kernel(table, ids2d)


def _tc_body(resp_ref, e_ref, wi_ref, wr_ref, b_ref, out_ref, *, k):
    sb = out_ref.shape[0]
    c = lax.broadcasted_iota(jnp.int32, (k, 1), 0).astype(jnp.float32)
    inv = 1.0 / (k - 1)
    bias = b_ref[...]
    wi = wi_ref[...]
    wr = wr_ref[...]
    for s in range(sb):
        es = e_ref[s]
        acc = lax.dot_general(wi, es, (((1,), (1,)), ((), ())),
                              preferred_element_type=jnp.float32)
        rf = resp_ref[s, :].astype(jnp.float32).reshape(1, -1)
        w = jnp.maximum(1.0 - jnp.abs(c - rf) * inv, 0.0)
        acc = acc + lax.dot_general(wr, w, (((1,), (0,)), ((), ())),
                                    preferred_element_type=jnp.float32)
        out_ref[s] = acc + bias


def _tc_project(e3d, resp_t, w_item, w_resp_w, bias2d, sb, bb):
    s, b, h = e3d.shape
    v = w_item.shape[0]
    k = w_resp_w.shape[1]
    grid = (b // bb, s // sb)
    out = pl.pallas_call(
        functools.partial(_tc_body, k=k),
        grid=grid,
        in_specs=[
            pl.BlockSpec((sb, bb), lambda j, i: (i, j)),
            pl.BlockSpec((sb, bb, h), lambda j, i: (i, j, 0)),
            pl.BlockSpec((v, h), lambda j, i: (0, 0)),
            pl.BlockSpec((v, k), lambda j, i: (0, 0)),
            pl.BlockSpec((v, 1), lambda j, i: (0, 0)),
        ],
        out_specs=pl.BlockSpec((sb, v, bb), lambda j, i: (i, 0, j)),
        out_shape=jax.ShapeDtypeStruct((s, v, b), jnp.float32),
    )(resp_t, e3d, w_item, w_resp_w, bias2d)
    return jnp.transpose(out, (2, 0, 1))


def kernel(question_ids, responses, item_embed, W_item, W_resp_w, W_resp_b):
    b, s = question_ids.shape
    n = b * s
    q1, h = item_embed.shape
    v = W_item.shape[0]
    k = W_resp_w.shape[1]

    ids3d = question_ids.T.reshape(_NW, n // (_NW * _G), _G).astype(jnp.int32)
    e = _sc_gather(item_embed, ids3d, n, h)

    return _tc_project(e.reshape(s, b, h), responses.T.astype(jnp.int32),
                       W_item, W_resp_w, W_resp_b.reshape(v, 1),
                       sb=8, bb=128)


# speedup vs baseline: 23.4226x; 1.6706x over previous
import functools

import jax
import jax.numpy as jnp
from jax import lax
from jax.experimental import pallas as pl
from jax.experimental.pallas import tpu as pltpu
from jax.experimental.pallas import tpu_sc as plsc

_NC = 2
_NS = 16
_NW = _NC * _NS

_G = 128
_KG = 5


def _sc_gather(table, ids2d, n_rows, h):
    rows_per_w = n_rows // _NW
    rows_per_it = _KG * _G
    n_it = rows_per_w // rows_per_it
    assert rows_per_w % rows_per_it == 0
    id_rows_per_w = rows_per_w // _G

    mesh = plsc.VectorSubcoreMesh(
        core_axis_name="c", subcore_axis_name="s",
        num_cores=_NC, num_subcores=_NS)

    @functools.partial(
        pl.kernel,
        out_type=jax.ShapeDtypeStruct((n_rows, h), jnp.float32),
        mesh=mesh,
        scratch_types=[
            pltpu.VMEM((id_rows_per_w, _G), jnp.int32),
            pltpu.VMEM((rows_per_it, h), jnp.float32),
            pltpu.SemaphoreType.DMA,
        ],
    )
    def gather_kernel(table_hbm, ids_hbm, out_hbm, idx_v, rows_v, sem):
        wid = lax.axis_index("s") * _NC + lax.axis_index("c")
        row0 = wid * rows_per_w

        pltpu.sync_copy(ids_hbm.at[wid], idx_v)

        def body(g, carry):
            copies = [
                pltpu.async_copy(
                    table_hbm.at[idx_v.at[g * _KG + j]],
                    rows_v.at[pl.ds(j * _G, _G)],
                    sem)
                for j in range(_KG)
            ]
            for c in copies:
                c.wait()
            pltpu.sync_copy(
                rows_v, out_hbm.at[pl.ds(row0 + g * rows_per_it, rows_per_it)])
            return carry

        lax.fori_loop(0, n_it, body, 0)

    return gather_kernel(table, ids2d)


def _tc_body(resp_ref, e_ref, wi_ref, wr_ref, b_ref, out_ref, *, k):
    sb = out_ref.shape[0]
    c = lax.broadcasted_iota(jnp.int32, (k, 1), 0).astype(jnp.float32)
    inv = 1.0 / (k - 1)
    bias = b_ref[...]
    wi = wi_ref[...]
    wr = wr_ref[...]
    for s in range(sb):
        es = e_ref[s]
        acc = lax.dot_general(wi, es, (((1,), (1,)), ((), ())),
                              preferred_element_type=jnp.float32)
        rf = resp_ref[s, :].astype(jnp.float32).reshape(1, -1)
        w = jnp.maximum(1.0 - jnp.abs(c - rf) * inv, 0.0)
        acc = acc + lax.dot_general(wr, w, (((1,), (0,)), ((), ())),
                                    preferred_element_type=jnp.float32)
        out_ref[s] = acc + bias


def _tc_project(e3d, resp_t, w_item, w_resp_w, bias2d, sb, bb):
    s, b, h = e3d.shape
    v = w_item.shape[0]
    k = w_resp_w.shape[1]
    grid = (b // bb, s // sb)
    out = pl.pallas_call(
        functools.partial(_tc_body, k=k),
        grid=grid,
        in_specs=[
            pl.BlockSpec((sb, bb), lambda j, i: (i, j)),
            pl.BlockSpec((sb, bb, h), lambda j, i: (i, j, 0)),
            pl.BlockSpec((v, h), lambda j, i: (0, 0)),
            pl.BlockSpec((v, k), lambda j, i: (0, 0)),
            pl.BlockSpec((v, 1), lambda j, i: (0, 0)),
        ],
        out_specs=pl.BlockSpec((sb, v, bb), lambda j, i: (i, 0, j)),
        out_shape=jax.ShapeDtypeStruct((s, v, b), jnp.float32),
    )(resp_t, e3d, w_item, w_resp_w, bias2d)
    return jnp.transpose(out, (2, 0, 1))


def kernel(question_ids, responses, item_embed, W_item, W_resp_w, W_resp_b):
    b, s = question_ids.shape
    n = b * s
    q1, h = item_embed.shape
    v = W_item.shape[0]
    k = W_resp_w.shape[1]

    ids3d = question_ids.T.reshape(_NW, n // (_NW * _G), _G).astype(jnp.int32)
    e = _sc_gather(item_embed, ids3d, n, h)

    return _tc_project(e.reshape(s, b, h), responses.T.astype(jnp.int32),
                       W_item, W_resp_w, W_resp_b.reshape(v, 1),
                       sb=8, bb=1024)


# speedup vs baseline: 23.7394x; 1.0135x over previous
import functools

import jax
import jax.numpy as jnp
from jax import lax
from jax.experimental import pallas as pl
from jax.experimental.pallas import tpu as pltpu
from jax.experimental.pallas import tpu_sc as plsc

_NC = 2
_NS = 16
_NW = _NC * _NS

_G = 128
_KG = 2


def _sc_gather(table, ids2d, n_rows, h):
    rows_per_w = n_rows // _NW
    rows_per_it = _KG * _G
    n_it = rows_per_w // rows_per_it
    assert rows_per_w % rows_per_it == 0
    id_rows_per_w = rows_per_w // _G

    mesh = plsc.VectorSubcoreMesh(
        core_axis_name="c", subcore_axis_name="s",
        num_cores=_NC, num_subcores=_NS)

    @functools.partial(
        pl.kernel,
        out_type=jax.ShapeDtypeStruct((n_rows, h), jnp.float32),
        mesh=mesh,
        scratch_types=[
            pltpu.VMEM((id_rows_per_w, _G), jnp.int32),
            pltpu.VMEM((2 * rows_per_it, h), jnp.float32),
            pltpu.SemaphoreType.DMA,
            pltpu.SemaphoreType.DMA,
        ],
    )
    def gather_kernel(table_hbm, ids_hbm, out_hbm, idx_v, rows_v, sem_g, sem_w):
        wid = lax.axis_index("s") * _NC + lax.axis_index("c")
        row0 = wid * rows_per_w

        pltpu.sync_copy(ids_hbm.at[wid], idx_v)

        def body(g, carry):
            half = (g % 2) * rows_per_it
            @pl.when(g >= 2)
            def _drain():
                pltpu.make_async_copy(
                    rows_v.at[pl.ds(half, rows_per_it)],
                    out_hbm.at[pl.ds(row0, rows_per_it)],
                    sem_w).wait()

            copies = [
                pltpu.async_copy(
                    table_hbm.at[idx_v.at[g * _KG + j]],
                    rows_v.at[pl.ds(half + j * _G, _G)],
                    sem_g)
                for j in range(_KG)
            ]
            for c in copies:
                c.wait()
            pltpu.async_copy(
                rows_v.at[pl.ds(half, rows_per_it)],
                out_hbm.at[pl.ds(row0 + g * rows_per_it, rows_per_it)],
                sem_w)
            return carry

        lax.fori_loop(0, n_it, body, 0)
        for _ in range(2):
            pltpu.make_async_copy(
                rows_v.at[pl.ds(0, rows_per_it)],
                out_hbm.at[pl.ds(row0, rows_per_it)],
                sem_w).wait()

    return gather_kernel(table, ids2d)


def _tc_body(resp_ref, e_ref, wi_ref, wr_ref, b_ref, out_ref, *, k):
    sb = out_ref.shape[0]
    c = lax.broadcasted_iota(jnp.int32, (k, 1), 0).astype(jnp.float32)
    inv = 1.0 / (k - 1)
    bias = b_ref[...]
    wi = wi_ref[...]
    wr = wr_ref[...]
    for s in range(sb):
        es = e_ref[s]
        acc = lax.dot_general(wi, es, (((1,), (1,)), ((), ())),
                              preferred_element_type=jnp.float32)
        rf = resp_ref[s, :].astype(jnp.float32).reshape(1, -1)
        w = jnp.maximum(1.0 - jnp.abs(c - rf) * inv, 0.0)
        acc = acc + lax.dot_general(wr, w, (((1,), (0,)), ((), ())),
                                    preferred_element_type=jnp.float32)
        out_ref[s] = acc + bias


def _tc_project(e3d, resp_t, w_item, w_resp_w, bias2d, sb, bb):
    s, b, h = e3d.shape
    v = w_item.shape[0]
    k = w_resp_w.shape[1]
    grid = (b // bb, s // sb)
    out = pl.pallas_call(
        functools.partial(_tc_body, k=k),
        grid=grid,
        in_specs=[
            pl.BlockSpec((sb, bb), lambda j, i: (i, j)),
            pl.BlockSpec((sb, bb, h), lambda j, i: (i, j, 0)),
            pl.BlockSpec((v, h), lambda j, i: (0, 0)),
            pl.BlockSpec((v, k), lambda j, i: (0, 0)),
            pl.BlockSpec((v, 1), lambda j, i: (0, 0)),
        ],
        out_specs=pl.BlockSpec((sb, v, bb), lambda j, i: (i, 0, j)),
        out_shape=jax.ShapeDtypeStruct((s, v, b), jnp.float32),
    )(resp_t, e3d, w_item, w_resp_w, bias2d)
    return jnp.transpose(out, (2, 0, 1))


def kernel(question_ids, responses, item_embed, W_item, W_resp_w, W_resp_b):
    b, s = question_ids.shape
    n = b * s
    q1, h = item_embed.shape
    v = W_item.shape[0]
    k = W_resp_w.shape[1]

    ids3d = question_ids.T.reshape(_NW, n // (_NW * _G), _G).astype(jnp.int32)
    e = _sc_gather(item_embed, ids3d, n, h)

    return _tc_project(e.reshape(s, b, h), responses.T.astype(jnp.int32),
                       W_item, W_resp_w, W_resp_b.reshape(v, 1),
                       sb=8, bb=1024)
